# Initial kernel scaffold; baseline (speedup 1.0000x reference)
import functools

import jax
import jax.numpy as jnp
from jax import lax
from jax.experimental import pallas as pl
from jax.experimental.pallas import tpu as pltpu
from jax.experimental.pallas import tpu_sc as plsc

NN = 10000
NP = 10240
EE = 320000
NC = 2
NS = 16
NW = NC * NS
KB = 128
NB = 79
EP = NW * NB * KB
DA = 144
DF = 80
RPT = NP // NS

_mesh = plsc.VectorSubcoreMesh(
    core_axis_name="c", subcore_axis_name="s", num_cores=NC, num_subcores=NS
)


def _zero_rows(rows_v, width):
    zero = jnp.zeros((16,), jnp.float32)

    def zr(r, carry):
        for c in range(width // 16):
            rows_v[r, pl.ds(c * 16, 16)] = zero
        return carry

    lax.fori_loop(0, KB, zr, 0)


def _pass1_body(x_hbm, src_hbm, dst_hbm, out_hbm, src_v, dst_v, rows_v, acc_sh):
    cid = lax.axis_index("c")
    sid = lax.axis_index("s")
    wid = sid * NC + cid
    _zero_rows(rows_v, DA)
    for b in range(RPT // KB):
        pltpu.sync_copy(rows_v, acc_sh.at[pl.ds(sid * RPT + b * KB, KB)])
    plsc.subcore_barrier()
    pltpu.sync_copy(src_hbm.at[wid], src_v)
    pltpu.sync_copy(dst_hbm.at[wid], dst_v)

    def eb(j, carry):
        pltpu.sync_copy(x_hbm.at[src_v.at[j]], rows_v)
        pltpu.sync_copy(rows_v, acc_sh.at[dst_v.at[j]], add=True)
        return carry

    lax.fori_loop(0, NB, eb, 0)
    plsc.subcore_barrier()
    pltpu.sync_copy(
        acc_sh.at[pl.ds(sid * RPT, RPT)], out_hbm.at[cid].at[pl.ds(sid * RPT, RPT)]
    )


_pass1 = pl.kernel(
    _pass1_body,
    out_type=jax.ShapeDtypeStruct((NC, NP, DA), jnp.float32),
    mesh=_mesh,
    scratch_types=[
        pltpu.VMEM((NB, KB), jnp.int32),
        pltpu.VMEM((NB, KB), jnp.int32),
        pltpu.VMEM((KB, DA), jnp.float32),
        pltpu.VMEM_SHARED((NP, DA), jnp.float32),
    ],
)


def _pass2_body(
    feat_hbm, el_hbm, er_hbm, src_hbm, dst_hbm, out_hbm,
    src_v, dst_v, el_tab, er_tab, rows_v, ee_v, acc_sh,
):
    cid = lax.axis_index("c")
    sid = lax.axis_index("s")
    wid = sid * NC + cid
    _zero_rows(rows_v, DF)
    for b in range(RPT // KB):
        pltpu.sync_copy(rows_v, acc_sh.at[pl.ds(sid * RPT + b * KB, KB)])
    plsc.subcore_barrier()
    pltpu.sync_copy(src_hbm.at[wid], src_v)
    pltpu.sync_copy(dst_hbm.at[wid], dst_v)
    pltpu.sync_copy(el_hbm, el_tab)
    pltpu.sync_copy(er_hbm, er_tab)

    def eb(j, carry):
        pltpu.sync_copy(feat_hbm.at[src_v.at[j]], rows_v)

        def ce(g, c2):
            s16 = src_v[j, pl.ds(g * 16, 16)]
            d16 = dst_v[j, pl.ds(g * 16, 16)]
            s = plsc.load_gather(el_tab, [s16]) + plsc.load_gather(er_tab, [d16])
            ee_v[pl.ds(g * 16, 16)] = jnp.exp(jnp.maximum(s, 0.2 * s))
            return c2

        lax.fori_loop(0, KB // 16, ce, 0)

        def sr(r, c2):
            s = ee_v[r]
            for c in range(DF // 16):
                rows_v[r, pl.ds(c * 16, 16)] = rows_v[r, pl.ds(c * 16, 16)] * s
            return c2

        lax.fori_loop(0, KB, sr, 0)
        pltpu.sync_copy(rows_v, acc_sh.at[dst_v.at[j]], add=True)
        return carry

    lax.fori_loop(0, NB, eb, 0)
    plsc.subcore_barrier()
    pltpu.sync_copy(
        acc_sh.at[pl.ds(sid * RPT, RPT)], out_hbm.at[cid].at[pl.ds(sid * RPT, RPT)]
    )


_pass2 = pl.kernel(
    _pass2_body,
    out_type=jax.ShapeDtypeStruct((NC, NP, DF), jnp.float32),
    mesh=_mesh,
    scratch_types=[
        pltpu.VMEM((NB, KB), jnp.int32),
        pltpu.VMEM((NB, KB), jnp.int32),
        pltpu.VMEM((NP,), jnp.float32),
        pltpu.VMEM((NP,), jnp.float32),
        pltpu.VMEM((KB, DF), jnp.float32),
        pltpu.VMEM((KB,), jnp.float32),
        pltpu.VMEM_SHARED((NP, DF), jnp.float32),
    ],
)

_BLK = 256
_HI = jax.lax.Precision.HIGHEST


def _dense_body(
    x_ref, a0_ref, a1_ref, ws_ref, wn_ref, bs_ref, wg_ref, al_ref, ar_ref,
    wr_ref, bg_ref, fa_ref, el_ref, er_ref, rest_ref,
):
    agg = a0_ref[...] + a1_ref[...]
    deg = jnp.maximum(agg[:, 128:129], 1.0)
    hn = agg[:, :128] / deg
    h = (
        jnp.dot(x_ref[...], ws_ref[...], precision=_HI)
        + jnp.dot(hn, wn_ref[...], precision=_HI)
        + bs_ref[...]
    )
    h = jnp.where(h > 0, h, jnp.expm1(h))
    feat = jnp.dot(h, wg_ref[...], precision=_HI)
    el_ref[...] = jnp.sum(feat * al_ref[...], axis=1, keepdims=True)
    er_ref[...] = jnp.sum(feat * ar_ref[...], axis=1, keepdims=True)
    rest_ref[...] = jnp.dot(h, wr_ref[...], precision=_HI) + bg_ref[...]
    fa_ref[...] = jnp.concatenate(
        [
            feat,
            jnp.ones((_BLK, 1), jnp.float32),
            jnp.zeros((_BLK, DF - 65), jnp.float32),
        ],
        axis=1,
    )


def _dense(x_pad, agg0, agg1, ws, wn, bs, wg, al, ar, wr, bg):
    g = NP // _BLK
    full = lambda shp: pl.BlockSpec(shp, lambda i: (0, 0))
    blk = lambda w: pl.BlockSpec((_BLK, w), lambda i: (i, 0))
    return pl.pallas_call(
        _dense_body,
        grid=(g,),
        in_specs=[
            blk(128), blk(DA), blk(DA),
            full((128, 128)), full((128, 128)), full((1, 128)),
            full((128, 64)), full((1, 64)), full((1, 64)),
            full((128, 64)), full((1, 64)),
        ],
        out_specs=[blk(DF), blk(1), blk(1), blk(64)],
        out_shape=[
            jax.ShapeDtypeStruct((NP, DF), jnp.float32),
            jax.ShapeDtypeStruct((NP, 1), jnp.float32),
            jax.ShapeDtypeStruct((NP, 1), jnp.float32),
            jax.ShapeDtypeStruct((NP, 64), jnp.float32),
        ],
    )(x_pad, agg0, agg1, ws, wn, bs, wg, al, ar, wr, bg)


def _final_body(n0_ref, n1_ref, rest_ref, o_ref):
    nm = n0_ref[...] + n1_ref[...]
    o_ref[...] = nm[:, :64] / (nm[:, 64:65] + 1e-9) + rest_ref[...]


def _final(n0, n1, rest):
    g = NP // _BLK
    blk = lambda w: pl.BlockSpec((_BLK, w), lambda i: (i, 0))
    return pl.pallas_call(
        _final_body,
        grid=(g,),
        in_specs=[blk(DF), blk(DF), blk(64)],
        out_specs=blk(64),
        out_shape=jax.ShapeDtypeStruct((NP, 64), jnp.float32),
    )(n0, n1, rest)


def kernel(x, edge_index, W_self, W_neigh, b_sage, W_gat, attn_l, attn_r, W_res, b_gat):
    src = edge_index[0]
    dst = edge_index[1]
    pad = jnp.full((EP - EE,), NP - 1, jnp.int32)
    srcp = jnp.concatenate([src, pad]).reshape(NW, NB, KB)
    dstp = jnp.concatenate([dst, pad]).reshape(NW, NB, KB)

    x_pad = jnp.pad(x, ((0, NP - NN), (0, 0)))
    x_aug = jnp.concatenate(
        [
            x_pad,
            jnp.ones((NP, 1), jnp.float32),
            jnp.zeros((NP, DA - 129), jnp.float32),
        ],
        axis=1,
    )

    aggp = _pass1(x_aug, srcp, dstp)

    feat_aug, el2, er2, rest = _dense(
        x_pad,
        aggp[0],
        aggp[1],
        W_self,
        W_neigh,
        b_sage.reshape(1, 128),
        W_gat,
        attn_l.reshape(1, 64),
        attn_r.reshape(1, 64),
        W_res,
        b_gat.reshape(1, 64),
    )

    nume = _pass2(feat_aug, el2.reshape(NP), er2.reshape(NP), srcp, dstp)

    out = _final(nume[0], nume[1], rest)
    return out[:NN]


# speedup vs baseline: 15.7069x; 15.7069x over previous
import functools

import jax
import jax.numpy as jnp
from jax import lax
from jax.experimental import pallas as pl
from jax.experimental.pallas import tpu as pltpu
from jax.experimental.pallas import tpu_sc as plsc

NN = 10000
NP = 10240
EE = 320000
NC = 2
NS = 16
NW = NC * NS
KB = 128
NB = 79
EP = NW * NB * KB
DA = 144
DF = 80
RPT = NP // NS

_mesh = plsc.VectorSubcoreMesh(
    core_axis_name="c", subcore_axis_name="s", num_cores=NC, num_subcores=NS
)
_sc_params = pltpu.CompilerParams(
    use_tc_tiling_on_sc=False, needs_layout_passes=False
)


def _zero_rows(rows_v, width):
    zero = jnp.zeros((16,), jnp.float32)

    def zr(r, carry):
        for c in range(width // 16):
            rows_v[r, pl.ds(c * 16, 16)] = zero
        return carry

    lax.fori_loop(0, KB, zr, 0)


def _pass1_body(x_hbm, src_hbm, dst_hbm, out_hbm, src_v, dst_v, rows_v, acc_sh):
    cid = lax.axis_index("c")
    sid = lax.axis_index("s")
    wid = sid * NC + cid
    _zero_rows(rows_v, DA)
    for b in range(RPT // KB):
        pltpu.sync_copy(rows_v, acc_sh.at[pl.ds(sid * RPT + b * KB, KB)])
    plsc.subcore_barrier()
    pltpu.sync_copy(src_hbm.at[wid], src_v)
    pltpu.sync_copy(dst_hbm.at[wid], dst_v)

    def eb(j, carry):
        pltpu.sync_copy(x_hbm.at[src_v.at[j]], rows_v)
        pltpu.sync_copy(rows_v, acc_sh.at[dst_v.at[j]], add=True)
        return carry

    lax.fori_loop(0, NB, eb, 0)
    plsc.subcore_barrier()
    pltpu.sync_copy(
        acc_sh.at[pl.ds(sid * RPT, RPT)], out_hbm.at[cid].at[pl.ds(sid * RPT, RPT)]
    )


_pass1 = pl.kernel(
    _pass1_body,
    out_type=jax.ShapeDtypeStruct((NC, NP, DA), jnp.float32),
    mesh=_mesh,
    scratch_types=[
        pltpu.VMEM((NB, KB), jnp.int32),
        pltpu.VMEM((NB, KB), jnp.int32),
        pltpu.VMEM((KB, DA), jnp.float32),
        pltpu.VMEM_SHARED((NP, DA), jnp.float32),
    ],
    compiler_params=_sc_params,
)


def _pass2_body(
    feat_hbm, el_hbm, er_hbm, src_hbm, dst_hbm, out_hbm,
    src_v, dst_v, el_tab, er_tab, rows_v, acc_sh,
):
    cid = lax.axis_index("c")
    sid = lax.axis_index("s")
    wid = sid * NC + cid
    _zero_rows(rows_v, DF)
    for b in range(RPT // KB):
        pltpu.sync_copy(rows_v, acc_sh.at[pl.ds(sid * RPT + b * KB, KB)])
    plsc.subcore_barrier()
    pltpu.sync_copy(src_hbm.at[wid], src_v)
    pltpu.sync_copy(dst_hbm.at[wid], dst_v)
    pltpu.sync_copy(el_hbm, el_tab)
    pltpu.sync_copy(er_hbm, er_tab)

    def eb(j, carry):
        pltpu.sync_copy(feat_hbm.at[src_v.at[j]], rows_v)

        def ce(g, c2):
            s16 = src_v[j, pl.ds(g * 16, 16)]
            d16 = dst_v[j, pl.ds(g * 16, 16)]
            s = plsc.load_gather(el_tab, [s16]) + plsc.load_gather(er_tab, [d16])
            ee16 = jnp.exp(jnp.maximum(s, 0.2 * s))
            for l in range(16):
                sc = ee16[l]
                r = g * 16 + l
                for c in range(DF // 16):
                    rows_v[r, pl.ds(c * 16, 16)] = rows_v[r, pl.ds(c * 16, 16)] * sc
            return c2

        lax.fori_loop(0, KB // 16, ce, 0)
        pltpu.sync_copy(rows_v, acc_sh.at[dst_v.at[j]], add=True)
        return carry

    lax.fori_loop(0, NB, eb, 0)
    plsc.subcore_barrier()
    pltpu.sync_copy(
        acc_sh.at[pl.ds(sid * RPT, RPT)], out_hbm.at[cid].at[pl.ds(sid * RPT, RPT)]
    )


_pass2 = pl.kernel(
    _pass2_body,
    out_type=jax.ShapeDtypeStruct((NC, NP, DF), jnp.float32),
    mesh=_mesh,
    scratch_types=[
        pltpu.VMEM((NB, KB), jnp.int32),
        pltpu.VMEM((NB, KB), jnp.int32),
        pltpu.VMEM((NP,), jnp.float32),
        pltpu.VMEM((NP,), jnp.float32),
        pltpu.VMEM((KB, DF), jnp.float32),
        pltpu.VMEM_SHARED((NP, DF), jnp.float32),
    ],
    compiler_params=_sc_params,
)

_BLK = 256
_HI = jax.lax.Precision.HIGHEST


def _dense_body(
    x_ref, a0_ref, a1_ref, ws_ref, wn_ref, bs_ref, wg_ref, al_ref, ar_ref,
    wr_ref, bg_ref, fa_ref, el_ref, er_ref, rest_ref,
):
    agg = a0_ref[...] + a1_ref[...]
    deg = jnp.maximum(agg[:, 128:129], 1.0)
    hn = agg[:, :128] / deg
    h = (
        jnp.dot(x_ref[...], ws_ref[...], precision=_HI)
        + jnp.dot(hn, wn_ref[...], precision=_HI)
        + bs_ref[...]
    )
    h = jnp.where(h > 0, h, jnp.exp(h) - 1.0)
    feat = jnp.dot(h, wg_ref[...], precision=_HI)
    el_ref[...] = jnp.sum(feat * al_ref[...], axis=1, keepdims=True)
    er_ref[...] = jnp.sum(feat * ar_ref[...], axis=1, keepdims=True)
    rest_ref[...] = jnp.dot(h, wr_ref[...], precision=_HI) + bg_ref[...]
    fa_ref[...] = jnp.concatenate(
        [
            feat,
            jnp.ones((_BLK, 1), jnp.float32),
            jnp.zeros((_BLK, DF - 65), jnp.float32),
        ],
        axis=1,
    )


def _dense(x_pad, agg0, agg1, ws, wn, bs, wg, al, ar, wr, bg):
    g = NP // _BLK
    full = lambda shp: pl.BlockSpec(shp, lambda i: (0, 0))
    blk = lambda w: pl.BlockSpec((_BLK, w), lambda i: (i, 0))
    return pl.pallas_call(
        _dense_body,
        grid=(g,),
        in_specs=[
            blk(128), blk(DA), blk(DA),
            full((128, 128)), full((128, 128)), full((1, 128)),
            full((128, 64)), full((1, 64)), full((1, 64)),
            full((128, 64)), full((1, 64)),
        ],
        out_specs=[blk(DF), blk(1), blk(1), blk(64)],
        out_shape=[
            jax.ShapeDtypeStruct((NP, DF), jnp.float32),
            jax.ShapeDtypeStruct((NP, 1), jnp.float32),
            jax.ShapeDtypeStruct((NP, 1), jnp.float32),
            jax.ShapeDtypeStruct((NP, 64), jnp.float32),
        ],
    )(x_pad, agg0, agg1, ws, wn, bs, wg, al, ar, wr, bg)


def _final_body(n0_ref, n1_ref, rest_ref, o_ref):
    nm = n0_ref[...] + n1_ref[...]
    o_ref[...] = nm[:, :64] / (nm[:, 64:65] + 1e-9) + rest_ref[...]


def _final(n0, n1, rest):
    g = NP // _BLK
    blk = lambda w: pl.BlockSpec((_BLK, w), lambda i: (i, 0))
    return pl.pallas_call(
        _final_body,
        grid=(g,),
        in_specs=[blk(DF), blk(DF), blk(64)],
        out_specs=blk(64),
        out_shape=jax.ShapeDtypeStruct((NP, 64), jnp.float32),
    )(n0, n1, rest)


def kernel(x, edge_index, W_self, W_neigh, b_sage, W_gat, attn_l, attn_r, W_res, b_gat):
    src = edge_index[0]
    dst = edge_index[1]
    pad = jnp.full((EP - EE,), NP - 1, jnp.int32)
    srcp = jnp.concatenate([src, pad]).reshape(NW, NB, KB)
    dstp = jnp.concatenate([dst, pad]).reshape(NW, NB, KB)

    x_pad = jnp.pad(x, ((0, NP - NN), (0, 0)))
    x_aug = jnp.concatenate(
        [
            x_pad,
            jnp.ones((NP, 1), jnp.float32),
            jnp.zeros((NP, DA - 129), jnp.float32),
        ],
        axis=1,
    )

    aggp = _pass1(x_aug, srcp, dstp)

    feat_aug, el2, er2, rest = _dense(
        x_pad,
        aggp[0],
        aggp[1],
        W_self,
        W_neigh,
        b_sage.reshape(1, 128),
        W_gat,
        attn_l.reshape(1, 64),
        attn_r.reshape(1, 64),
        W_res,
        b_gat.reshape(1, 64),
    )

    nume = _pass2(feat_aug, el2.reshape(NP), er2.reshape(NP), srcp, dstp)

    out = _final(nume[0], nume[1], rest)
    return out[:NN]


# speedup vs baseline: 17.9712x; 1.1442x over previous
import functools

import jax
import jax.numpy as jnp
from jax import lax
from jax.experimental import pallas as pl
from jax.experimental.pallas import tpu as pltpu
from jax.experimental.pallas import tpu_sc as plsc

NN = 10000
NP = 10240
EE = 320000
NC = 2
NS = 16
NW = NC * NS
KB1 = 64
NB1 = 158
KB2 = 128
NB2 = 79
EP = NW * NB1 * KB1
DA = 136
DF = 80
RPT = NP // NS

_mesh = plsc.VectorSubcoreMesh(
    core_axis_name="c", subcore_axis_name="s", num_cores=NC, num_subcores=NS
)
_sc_params = pltpu.CompilerParams(
    use_tc_tiling_on_sc=False, needs_layout_passes=False
)


def _zero_rows(rows_v, nrows, width):
    zero = jnp.zeros((16,), jnp.float32)

    def zr(r, carry):
        for c in range(width // 16):
            rows_v[r, pl.ds(c * 16, 16)] = zero
        return carry

    lax.fori_loop(0, nrows, zr, 0)


def _pass1_body(
    x_hbm, src_hbm, dst_hbm, out_hbm, src_v, dst_v, rows_v, acc_sh, isem, gsem
):
    cid = lax.axis_index("c")
    sid = lax.axis_index("s")
    wid = sid * NC + cid
    pltpu.async_copy(src_hbm.at[wid], src_v, isem)
    pltpu.async_copy(dst_hbm.at[wid], dst_v, isem)
    _zero_rows(rows_v.at[0], KB1, DA)
    for b in range(RPT // KB1):
        pltpu.sync_copy(rows_v.at[0], acc_sh.at[pl.ds(sid * RPT + b * KB1, KB1)])
    pltpu.make_async_copy(src_hbm.at[wid], src_v, isem).wait()
    pltpu.make_async_copy(dst_hbm.at[wid], dst_v, isem).wait()
    plsc.subcore_barrier()
    pltpu.async_copy(x_hbm.at[src_v.at[0]], rows_v.at[0], gsem)

    def eb(j, carry):
        b = lax.rem(j, 2)
        pltpu.make_async_copy(x_hbm.at[src_v.at[j]], rows_v.at[b], gsem).wait()

        @pl.when(j < NB1 - 1)
        def _():
            pltpu.async_copy(x_hbm.at[src_v.at[j + 1]], rows_v.at[1 - b], gsem)

        pltpu.sync_copy(rows_v.at[b], acc_sh.at[dst_v.at[j]], add=True)
        return carry

    lax.fori_loop(0, NB1, eb, 0)
    plsc.subcore_barrier()
    pltpu.sync_copy(
        acc_sh.at[pl.ds(sid * RPT, RPT)], out_hbm.at[cid].at[pl.ds(sid * RPT, RPT)]
    )


_pass1 = pl.kernel(
    _pass1_body,
    out_type=jax.ShapeDtypeStruct((NC, NP, DA), jnp.float32),
    mesh=_mesh,
    scratch_types=[
        pltpu.VMEM((NB1, KB1), jnp.int32),
        pltpu.VMEM((NB1, KB1), jnp.int32),
        pltpu.VMEM((2, KB1, DA), jnp.float32),
        pltpu.VMEM_SHARED((NP, DA), jnp.float32),
        pltpu.SemaphoreType.DMA,
        pltpu.SemaphoreType.DMA,
    ],
    compiler_params=_sc_params,
)


def _pass2_body(
    feat_hbm, el_hbm, er_hbm, src_hbm, dst_hbm, out_hbm,
    src_v, dst_v, el_tab, er_tab, rows_v, acc_sh, isem, gsem,
):
    cid = lax.axis_index("c")
    sid = lax.axis_index("s")
    wid = sid * NC + cid
    pltpu.async_copy(src_hbm.at[wid], src_v, isem)
    pltpu.async_copy(dst_hbm.at[wid], dst_v, isem)
    pltpu.async_copy(el_hbm, el_tab, isem)
    pltpu.async_copy(er_hbm, er_tab, isem)
    _zero_rows(rows_v.at[0], KB2, DF)
    for b in range(RPT // KB2):
        pltpu.sync_copy(rows_v.at[0], acc_sh.at[pl.ds(sid * RPT + b * KB2, KB2)])
    pltpu.make_async_copy(src_hbm.at[wid], src_v, isem).wait()
    pltpu.make_async_copy(dst_hbm.at[wid], dst_v, isem).wait()
    pltpu.make_async_copy(el_hbm, el_tab, isem).wait()
    pltpu.make_async_copy(er_hbm, er_tab, isem).wait()
    plsc.subcore_barrier()
    pltpu.async_copy(feat_hbm.at[src_v.at[0]], rows_v.at[0], gsem)

    def eb(j, carry):
        b = lax.rem(j, 2)
        pltpu.make_async_copy(feat_hbm.at[src_v.at[j]], rows_v.at[b], gsem).wait()

        @pl.when(j < NB2 - 1)
        def _():
            pltpu.async_copy(feat_hbm.at[src_v.at[j + 1]], rows_v.at[1 - b], gsem)

        def ce(g, c2):
            s16 = src_v[j, pl.ds(g * 16, 16)]
            d16 = dst_v[j, pl.ds(g * 16, 16)]
            s = plsc.load_gather(el_tab, [s16]) + plsc.load_gather(er_tab, [d16])
            ee16 = jnp.exp(jnp.maximum(s, 0.2 * s))
            for l in range(16):
                sc = ee16[l]
                r = g * 16 + l
                for c in range(DF // 16):
                    rows_v[b, r, pl.ds(c * 16, 16)] = (
                        rows_v[b, r, pl.ds(c * 16, 16)] * sc
                    )
            return c2

        lax.fori_loop(0, KB2 // 16, ce, 0)
        pltpu.sync_copy(rows_v.at[b], acc_sh.at[dst_v.at[j]], add=True)
        return carry

    lax.fori_loop(0, NB2, eb, 0)
    plsc.subcore_barrier()
    pltpu.sync_copy(
        acc_sh.at[pl.ds(sid * RPT, RPT)], out_hbm.at[cid].at[pl.ds(sid * RPT, RPT)]
    )


_pass2 = pl.kernel(
    _pass2_body,
    out_type=jax.ShapeDtypeStruct((NC, NP, DF), jnp.float32),
    mesh=_mesh,
    scratch_types=[
        pltpu.VMEM((NB2, KB2), jnp.int32),
        pltpu.VMEM((NB2, KB2), jnp.int32),
        pltpu.VMEM((NP,), jnp.float32),
        pltpu.VMEM((NP,), jnp.float32),
        pltpu.VMEM((2, KB2, DF), jnp.float32),
        pltpu.VMEM_SHARED((NP, DF), jnp.float32),
        pltpu.SemaphoreType.DMA,
        pltpu.SemaphoreType.DMA,
    ],
    compiler_params=_sc_params,
)

_BLK = 256
_HI = jax.lax.Precision.HIGHEST


def _dense_body(
    x_ref, a0_ref, a1_ref, ws_ref, wn_ref, bs_ref, wg_ref, al_ref, ar_ref,
    wr_ref, bg_ref, fa_ref, el_ref, er_ref, rest_ref,
):
    agg = a0_ref[...] + a1_ref[...]
    deg = jnp.maximum(agg[:, 128:129], 1.0)
    hn = agg[:, :128] / deg
    h = (
        jnp.dot(x_ref[...], ws_ref[...], precision=_HI)
        + jnp.dot(hn, wn_ref[...], precision=_HI)
        + bs_ref[...]
    )
    h = jnp.where(h > 0, h, jnp.exp(h) - 1.0)
    feat = jnp.dot(h, wg_ref[...], precision=_HI)
    el_ref[...] = jnp.sum(feat * al_ref[...], axis=1, keepdims=True)
    er_ref[...] = jnp.sum(feat * ar_ref[...], axis=1, keepdims=True)
    rest_ref[...] = jnp.dot(h, wr_ref[...], precision=_HI) + bg_ref[...]
    fa_ref[...] = jnp.concatenate(
        [
            feat,
            jnp.ones((_BLK, 1), jnp.float32),
            jnp.zeros((_BLK, DF - 65), jnp.float32),
        ],
        axis=1,
    )


def _dense(x_pad, agg0, agg1, ws, wn, bs, wg, al, ar, wr, bg):
    g = NP // _BLK
    full = lambda shp: pl.BlockSpec(shp, lambda i: (0, 0))
    blk = lambda w: pl.BlockSpec((_BLK, w), lambda i: (i, 0))
    return pl.pallas_call(
        _dense_body,
        grid=(g,),
        in_specs=[
            blk(128), blk(DA), blk(DA),
            full((128, 128)), full((128, 128)), full((1, 128)),
            full((128, 64)), full((1, 64)), full((1, 64)),
            full((128, 64)), full((1, 64)),
        ],
        out_specs=[blk(DF), blk(1), blk(1), blk(64)],
        out_shape=[
            jax.ShapeDtypeStruct((NP, DF), jnp.float32),
            jax.ShapeDtypeStruct((NP, 1), jnp.float32),
            jax.ShapeDtypeStruct((NP, 1), jnp.float32),
            jax.ShapeDtypeStruct((NP, 64), jnp.float32),
        ],
    )(x_pad, agg0, agg1, ws, wn, bs, wg, al, ar, wr, bg)


def _final_body(n0_ref, n1_ref, rest_ref, o_ref):
    nm = n0_ref[...] + n1_ref[...]
    o_ref[...] = nm[:, :64] / (nm[:, 64:65] + 1e-9) + rest_ref[...]


def _final(n0, n1, rest):
    g = NP // _BLK
    blk = lambda w: pl.BlockSpec((_BLK, w), lambda i: (i, 0))
    return pl.pallas_call(
        _final_body,
        grid=(g,),
        in_specs=[blk(DF), blk(DF), blk(64)],
        out_specs=blk(64),
        out_shape=jax.ShapeDtypeStruct((NP, 64), jnp.float32),
    )(n0, n1, rest)


def kernel(x, edge_index, W_self, W_neigh, b_sage, W_gat, attn_l, attn_r, W_res, b_gat):
    src = edge_index[0]
    dst = edge_index[1]
    pad = jnp.full((EP - EE,), NP - 1, jnp.int32)
    srcf = jnp.concatenate([src, pad])
    dstf = jnp.concatenate([dst, pad])
    srcp1 = srcf.reshape(NW, NB1, KB1)
    dstp1 = dstf.reshape(NW, NB1, KB1)
    srcp2 = srcf.reshape(NW, NB2, KB2)
    dstp2 = dstf.reshape(NW, NB2, KB2)

    x_pad = jnp.pad(x, ((0, NP - NN), (0, 0)))
    x_aug = jnp.concatenate(
        [
            x_pad,
            jnp.ones((NP, 1), jnp.float32),
            jnp.zeros((NP, DA - 129), jnp.float32),
        ],
        axis=1,
    )

    aggp = _pass1(x_aug, srcp1, dstp1)

    feat_aug, el2, er2, rest = _dense(
        x_pad,
        aggp[0],
        aggp[1],
        W_self,
        W_neigh,
        b_sage.reshape(1, 128),
        W_gat,
        attn_l.reshape(1, 64),
        attn_r.reshape(1, 64),
        W_res,
        b_gat.reshape(1, 64),
    )

    nume = _pass2(feat_aug, el2.reshape(NP), er2.reshape(NP), srcp2, dstp2)

    out = _final(nume[0], nume[1], rest)
    return out[:NN]


# speedup vs baseline: 22.1942x; 1.2350x over previous
import functools

import jax
import jax.numpy as jnp
from jax import lax
from jax.experimental import pallas as pl
from jax.experimental.pallas import tpu as pltpu
from jax.experimental.pallas import tpu_sc as plsc

NN = 10000
NP = 10240
EE = 320000
NC = 2
NS = 16
NW = NC * NS
KB1 = 64
NBA1 = 191
NBB1 = 122
R1ROWS = 16 * NBA1 + 16 * NBB1 + (NBA1 - NBB1)
KB2 = 128
NBA2 = 101
NBB2 = 56
R2ROWS = 16 * NBA2 + 16 * NBB2 + (NBA2 - NBB2)
EP = R2ROWS * KB2
DA = 136
DF = 80
RPT = NP // NS

_mesh = plsc.VectorSubcoreMesh(
    core_axis_name="c", subcore_axis_name="s", num_cores=NC, num_subcores=NS
)
_sc_params = pltpu.CompilerParams(
    use_tc_tiling_on_sc=False, needs_layout_passes=False
)


def _zero_rows(rows_v, nrows, width):
    zero = jnp.zeros((16,), jnp.float32)

    def zr(r, carry):
        for c in range(width // 16):
            rows_v[r, pl.ds(c * 16, 16)] = zero
        return carry

    lax.fori_loop(0, nrows, zr, 0)


def _pass1_body(
    x_hbm, src_hbm, dst_hbm, out_hbm, src_v, dst_v, rows_v, acc_sh, isem, gsem
):
    cid = lax.axis_index("c")
    sid = lax.axis_index("s")
    nb = jnp.where(cid == 0, NBA1, NBB1)
    base = jnp.where(cid == 0, sid * NBA1, 16 * NBA1 + sid * NBB1)
    pltpu.async_copy(src_hbm.at[pl.ds(base, NBA1)], src_v, isem)
    pltpu.async_copy(dst_hbm.at[pl.ds(base, NBA1)], dst_v, isem)
    _zero_rows(rows_v.at[0], KB1, DA)
    for b in range(RPT // KB1):
        pltpu.sync_copy(rows_v.at[0], acc_sh.at[pl.ds(sid * RPT + b * KB1, KB1)])
    pltpu.make_async_copy(src_hbm.at[pl.ds(base, NBA1)], src_v, isem).wait()
    pltpu.make_async_copy(dst_hbm.at[pl.ds(base, NBA1)], dst_v, isem).wait()
    plsc.subcore_barrier()
    pltpu.async_copy(x_hbm.at[src_v.at[0]], rows_v.at[0], gsem)

    def eb(j, carry):
        b = lax.rem(j, 2)
        pltpu.make_async_copy(x_hbm.at[src_v.at[j]], rows_v.at[b], gsem).wait()

        @pl.when(j < nb - 1)
        def _():
            pltpu.async_copy(x_hbm.at[src_v.at[j + 1]], rows_v.at[1 - b], gsem)

        pltpu.sync_copy(rows_v.at[b], acc_sh.at[dst_v.at[j]], add=True)
        return carry

    lax.fori_loop(0, nb, eb, 0)
    plsc.subcore_barrier()
    pltpu.sync_copy(
        acc_sh.at[pl.ds(sid * RPT, RPT)], out_hbm.at[cid].at[pl.ds(sid * RPT, RPT)]
    )


_pass1 = pl.kernel(
    _pass1_body,
    out_type=jax.ShapeDtypeStruct((NC, NP, DA), jnp.float32),
    mesh=_mesh,
    scratch_types=[
        pltpu.VMEM((NBA1, KB1), jnp.int32),
        pltpu.VMEM((NBA1, KB1), jnp.int32),
        pltpu.VMEM((2, KB1, DA), jnp.float32),
        pltpu.VMEM_SHARED((NP, DA), jnp.float32),
        pltpu.SemaphoreType.DMA,
        pltpu.SemaphoreType.DMA,
    ],
    compiler_params=_sc_params,
)


def _pass2_body(
    feat_hbm, el_hbm, er_hbm, src_hbm, dst_hbm, out_hbm,
    src_v, dst_v, el_tab, er_tab, rows_v, acc_sh, isem, gsem,
):
    cid = lax.axis_index("c")
    sid = lax.axis_index("s")
    nb = jnp.where(cid == 0, NBA2, NBB2)
    base = jnp.where(cid == 0, sid * NBA2, 16 * NBA2 + sid * NBB2)
    pltpu.async_copy(src_hbm.at[pl.ds(base, NBA2)], src_v, isem)
    pltpu.async_copy(dst_hbm.at[pl.ds(base, NBA2)], dst_v, isem)
    pltpu.async_copy(el_hbm, el_tab, isem)
    pltpu.async_copy(er_hbm, er_tab, isem)
    _zero_rows(rows_v.at[0], KB2, DF)
    for b in range(RPT // KB2):
        pltpu.sync_copy(rows_v.at[0], acc_sh.at[pl.ds(sid * RPT + b * KB2, KB2)])
    pltpu.make_async_copy(src_hbm.at[pl.ds(base, NBA2)], src_v, isem).wait()
    pltpu.make_async_copy(dst_hbm.at[pl.ds(base, NBA2)], dst_v, isem).wait()
    pltpu.make_async_copy(el_hbm, el_tab, isem).wait()
    pltpu.make_async_copy(er_hbm, er_tab, isem).wait()
    plsc.subcore_barrier()
    pltpu.async_copy(feat_hbm.at[src_v.at[0]], rows_v.at[0], gsem)

    def eb(j, carry):
        b = lax.rem(j, 2)
        pltpu.make_async_copy(feat_hbm.at[src_v.at[j]], rows_v.at[b], gsem).wait()

        @pl.when(j < nb - 1)
        def _():
            pltpu.async_copy(feat_hbm.at[src_v.at[j + 1]], rows_v.at[1 - b], gsem)

        def ce(g, c2):
            s16 = src_v[j, pl.ds(g * 16, 16)]
            d16 = dst_v[j, pl.ds(g * 16, 16)]
            s = plsc.load_gather(el_tab, [s16]) + plsc.load_gather(er_tab, [d16])
            ee16 = jnp.exp(jnp.maximum(s, 0.2 * s))
            for l in range(16):
                sc = ee16[l]
                r = g * 16 + l
                for c in range(DF // 16):
                    rows_v[b, r, pl.ds(c * 16, 16)] = (
                        rows_v[b, r, pl.ds(c * 16, 16)] * sc
                    )
            return c2

        lax.fori_loop(0, KB2 // 16, ce, 0)
        pltpu.sync_copy(rows_v.at[b], acc_sh.at[dst_v.at[j]], add=True)
        return carry

    lax.fori_loop(0, nb, eb, 0)
    plsc.subcore_barrier()
    pltpu.sync_copy(
        acc_sh.at[pl.ds(sid * RPT, RPT)], out_hbm.at[cid].at[pl.ds(sid * RPT, RPT)]
    )


_pass2 = pl.kernel(
    _pass2_body,
    out_type=jax.ShapeDtypeStruct((NC, NP, DF), jnp.float32),
    mesh=_mesh,
    scratch_types=[
        pltpu.VMEM((NBA2, KB2), jnp.int32),
        pltpu.VMEM((NBA2, KB2), jnp.int32),
        pltpu.VMEM((NP,), jnp.float32),
        pltpu.VMEM((NP,), jnp.float32),
        pltpu.VMEM((2, KB2, DF), jnp.float32),
        pltpu.VMEM_SHARED((NP, DF), jnp.float32),
        pltpu.SemaphoreType.DMA,
        pltpu.SemaphoreType.DMA,
    ],
    compiler_params=_sc_params,
)

_BLK = 256
_HI = jax.lax.Precision.HIGHEST


def _dense_body(
    x_ref, a0_ref, a1_ref, ws_ref, wn_ref, bs_ref, wg_ref, al_ref, ar_ref,
    wr_ref, bg_ref, fa_ref, el_ref, er_ref, rest_ref,
):
    agg = a0_ref[...] + a1_ref[...]
    deg = jnp.maximum(agg[:, 128:129], 1.0)
    hn = agg[:, :128] / deg
    h = (
        jnp.dot(x_ref[...], ws_ref[...], precision=_HI)
        + jnp.dot(hn, wn_ref[...], precision=_HI)
        + bs_ref[...]
    )
    h = jnp.where(h > 0, h, jnp.exp(h) - 1.0)
    feat = jnp.dot(h, wg_ref[...], precision=_HI)
    el_ref[...] = jnp.sum(feat * al_ref[...], axis=1, keepdims=True)
    er_ref[...] = jnp.sum(feat * ar_ref[...], axis=1, keepdims=True)
    rest_ref[...] = jnp.dot(h, wr_ref[...], precision=_HI) + bg_ref[...]
    fa_ref[...] = jnp.concatenate(
        [
            feat,
            jnp.ones((_BLK, 1), jnp.float32),
            jnp.zeros((_BLK, DF - 65), jnp.float32),
        ],
        axis=1,
    )


def _dense(x_pad, agg0, agg1, ws, wn, bs, wg, al, ar, wr, bg):
    g = NP // _BLK
    full = lambda shp: pl.BlockSpec(shp, lambda i: (0, 0))
    blk = lambda w: pl.BlockSpec((_BLK, w), lambda i: (i, 0))
    return pl.pallas_call(
        _dense_body,
        grid=(g,),
        in_specs=[
            blk(128), blk(DA), blk(DA),
            full((128, 128)), full((128, 128)), full((1, 128)),
            full((128, 64)), full((1, 64)), full((1, 64)),
            full((128, 64)), full((1, 64)),
        ],
        out_specs=[blk(DF), blk(1), blk(1), blk(64)],
        out_shape=[
            jax.ShapeDtypeStruct((NP, DF), jnp.float32),
            jax.ShapeDtypeStruct((NP, 1), jnp.float32),
            jax.ShapeDtypeStruct((NP, 1), jnp.float32),
            jax.ShapeDtypeStruct((NP, 64), jnp.float32),
        ],
    )(x_pad, agg0, agg1, ws, wn, bs, wg, al, ar, wr, bg)


def _final_body(n0_ref, n1_ref, rest_ref, o_ref):
    nm = n0_ref[...] + n1_ref[...]
    o_ref[...] = nm[:, :64] / (nm[:, 64:65] + 1e-9) + rest_ref[...]


def _final(n0, n1, rest):
    g = NP // _BLK
    blk = lambda w: pl.BlockSpec((_BLK, w), lambda i: (i, 0))
    return pl.pallas_call(
        _final_body,
        grid=(g,),
        in_specs=[blk(DF), blk(DF), blk(64)],
        out_specs=blk(64),
        out_shape=jax.ShapeDtypeStruct((NP, 64), jnp.float32),
    )(n0, n1, rest)


def kernel(x, edge_index, W_self, W_neigh, b_sage, W_gat, attn_l, attn_r, W_res, b_gat):
    src = edge_index[0]
    dst = edge_index[1]
    pad = jnp.full((EP - EE,), NP - 1, jnp.int32)
    srcf = jnp.concatenate([src, pad])
    dstf = jnp.concatenate([dst, pad])
    srcp1 = srcf[: R1ROWS * KB1].reshape(R1ROWS, KB1)
    dstp1 = dstf[: R1ROWS * KB1].reshape(R1ROWS, KB1)
    srcp2 = srcf.reshape(R2ROWS, KB2)
    dstp2 = dstf.reshape(R2ROWS, KB2)

    x_pad = jnp.pad(x, ((0, NP - NN), (0, 0)))
    x_aug = jnp.concatenate(
        [
            x_pad,
            jnp.ones((NP, 1), jnp.float32),
            jnp.zeros((NP, DA - 129), jnp.float32),
        ],
        axis=1,
    )

    aggp = _pass1(x_aug, srcp1, dstp1)

    feat_aug, el2, er2, rest = _dense(
        x_pad,
        aggp[0],
        aggp[1],
        W_self,
        W_neigh,
        b_sage.reshape(1, 128),
        W_gat,
        attn_l.reshape(1, 64),
        attn_r.reshape(1, 64),
        W_res,
        b_gat.reshape(1, 64),
    )

    nume = _pass2(feat_aug, el2.reshape(NP), er2.reshape(NP), srcp2, dstp2)

    out = _final(nume[0], nume[1], rest)
    return out[:NN]


# speedup vs baseline: 23.9119x; 1.0774x over previous
import functools

import jax
import jax.numpy as jnp
from jax import lax
from jax.experimental import pallas as pl
from jax.experimental.pallas import tpu as pltpu
from jax.experimental.pallas import tpu_sc as plsc

NN = 10000
NP = 10240
EE = 320000
NC = 2
NS = 16
NW = NC * NS
KB1 = 64
NBA1 = 191
NBB1 = 122
R1ROWS = 16 * NBA1 + 16 * NBB1 + (NBA1 - NBB1)
KB2 = 128
NBA2 = 101
NBB2 = 56
R2ROWS = 16 * NBA2 + 16 * NBB2 + (NBA2 - NBB2)
EP = R2ROWS * KB2
DA = 136
DF = 80
RPT = NP // NS

_mesh = plsc.VectorSubcoreMesh(
    core_axis_name="c", subcore_axis_name="s", num_cores=NC, num_subcores=NS
)
_sc_params = pltpu.CompilerParams(
    use_tc_tiling_on_sc=False, needs_layout_passes=False
)


def _zero_rows(rows_v, nrows, width):
    zero = jnp.zeros((16,), jnp.float32)

    def zr(r, carry):
        for c in range(width // 16):
            rows_v[r, pl.ds(c * 16, 16)] = zero
        return carry

    lax.fori_loop(0, nrows, zr, 0)


def _pass1_body(
    x_hbm, src_hbm, dst_hbm, out_hbm, src_v, dst_v, rows_v, acc_sh, isem, gsem
):
    cid = lax.axis_index("c")
    sid = lax.axis_index("s")
    nb = jnp.where(cid == 0, NBA1, NBB1)
    base = jnp.where(cid == 0, sid * NBA1, 16 * NBA1 + sid * NBB1)
    pltpu.async_copy(src_hbm.at[pl.ds(base, NBA1)], src_v, isem)
    pltpu.async_copy(dst_hbm.at[pl.ds(base, NBA1)], dst_v, isem)
    _zero_rows(rows_v.at[0], KB1, DA)
    for b in range(RPT // KB1):
        pltpu.sync_copy(rows_v.at[0], acc_sh.at[pl.ds(sid * RPT + b * KB1, KB1)])
    pltpu.make_async_copy(src_hbm.at[pl.ds(base, NBA1)], src_v, isem).wait()
    pltpu.make_async_copy(dst_hbm.at[pl.ds(base, NBA1)], dst_v, isem).wait()
    plsc.subcore_barrier()
    pltpu.async_copy(x_hbm.at[src_v.at[0]], rows_v.at[0], gsem)

    def eb(j, carry):
        b = lax.rem(j, 2)
        pltpu.make_async_copy(x_hbm.at[src_v.at[j]], rows_v.at[b], gsem).wait()

        @pl.when(j < nb - 1)
        def _():
            pltpu.async_copy(x_hbm.at[src_v.at[j + 1]], rows_v.at[1 - b], gsem)

        pltpu.sync_copy(rows_v.at[b], acc_sh.at[dst_v.at[j]], add=True)
        return carry

    lax.fori_loop(0, nb, eb, 0)
    plsc.subcore_barrier()
    pltpu.sync_copy(
        acc_sh.at[pl.ds(sid * RPT, RPT)], out_hbm.at[cid].at[pl.ds(sid * RPT, RPT)]
    )


_pass1 = pl.kernel(
    _pass1_body,
    out_type=jax.ShapeDtypeStruct((NC, NP, DA), jnp.float32),
    mesh=_mesh,
    scratch_types=[
        pltpu.VMEM((NBA1, KB1), jnp.int32),
        pltpu.VMEM((NBA1, KB1), jnp.int32),
        pltpu.VMEM((2, KB1, DA), jnp.float32),
        pltpu.VMEM_SHARED((NP, DA), jnp.float32),
        pltpu.SemaphoreType.DMA,
        pltpu.SemaphoreType.DMA,
    ],
    compiler_params=_sc_params,
)


def _pass2_body(
    feat_hbm, el_hbm, er_hbm, src_hbm, dst_hbm, out_hbm,
    src_v, dst_v, el_tab, er_tab, rows_v, acc_sh, isem, gsem,
):
    cid = lax.axis_index("c")
    sid = lax.axis_index("s")
    nb = jnp.where(cid == 0, NBA2, NBB2)
    base = jnp.where(cid == 0, sid * NBA2, 16 * NBA2 + sid * NBB2)
    pltpu.async_copy(src_hbm.at[pl.ds(base, NBA2)], src_v, isem)
    pltpu.async_copy(dst_hbm.at[pl.ds(base, NBA2)], dst_v, isem)
    pltpu.async_copy(el_hbm, el_tab, isem)
    pltpu.async_copy(er_hbm, er_tab, isem)
    _zero_rows(rows_v.at[0], KB2, DF)
    for b in range(RPT // KB2):
        pltpu.sync_copy(rows_v.at[0], acc_sh.at[pl.ds(sid * RPT + b * KB2, KB2)])
    pltpu.make_async_copy(src_hbm.at[pl.ds(base, NBA2)], src_v, isem).wait()
    pltpu.make_async_copy(dst_hbm.at[pl.ds(base, NBA2)], dst_v, isem).wait()
    pltpu.make_async_copy(el_hbm, el_tab, isem).wait()
    pltpu.make_async_copy(er_hbm, er_tab, isem).wait()
    plsc.subcore_barrier()
    pltpu.async_copy(feat_hbm.at[src_v.at[0]], rows_v.at[0], gsem)

    def eb(j, carry):
        b = lax.rem(j, 2)
        pltpu.make_async_copy(feat_hbm.at[src_v.at[j]], rows_v.at[b], gsem).wait()

        @pl.when(j < nb - 1)
        def _():
            pltpu.async_copy(feat_hbm.at[src_v.at[j + 1]], rows_v.at[1 - b], gsem)

        def ce(g, c2):
            s16 = src_v[j, pl.ds(g * 16, 16)]
            d16 = dst_v[j, pl.ds(g * 16, 16)]
            s = plsc.load_gather(el_tab, [s16]) + plsc.load_gather(er_tab, [d16])
            ee16 = jnp.exp(jnp.maximum(s, 0.2 * s))
            for l in range(16):
                sc = ee16[l]
                r = g * 16 + l
                for c in range(DF // 16):
                    rows_v[b, r, pl.ds(c * 16, 16)] = (
                        rows_v[b, r, pl.ds(c * 16, 16)] * sc
                    )
            return c2

        lax.fori_loop(0, KB2 // 16, ce, 0)
        pltpu.sync_copy(rows_v.at[b], acc_sh.at[dst_v.at[j]], add=True)
        return carry

    lax.fori_loop(0, nb, eb, 0)
    plsc.subcore_barrier()
    pltpu.sync_copy(
        acc_sh.at[pl.ds(sid * RPT, RPT)], out_hbm.at[cid].at[pl.ds(sid * RPT, RPT)]
    )


_pass2 = pl.kernel(
    _pass2_body,
    out_type=jax.ShapeDtypeStruct((NC, NP, DF), jnp.float32),
    mesh=_mesh,
    scratch_types=[
        pltpu.VMEM((NBA2, KB2), jnp.int32),
        pltpu.VMEM((NBA2, KB2), jnp.int32),
        pltpu.VMEM((NP,), jnp.float32),
        pltpu.VMEM((NP,), jnp.float32),
        pltpu.VMEM((2, KB2, DF), jnp.float32),
        pltpu.VMEM_SHARED((NP, DF), jnp.float32),
        pltpu.SemaphoreType.DMA,
        pltpu.SemaphoreType.DMA,
    ],
    compiler_params=_sc_params,
)

_BLK = 512
_BLKF = 400
_HI = jax.lax.Precision.HIGHEST


def _dense_body(
    x_ref, ap_ref, ws_ref, wn_ref, bs_ref, wg_ref, al_ref, ar_ref,
    wr_ref, bg_ref, fa_ref, el_ref, er_ref, rest_ref,
):
    agg = ap_ref[0] + ap_ref[1]
    deg = jnp.maximum(agg[:, 128:129], 1.0)
    hn = agg[:, :128] / deg
    h = (
        jnp.dot(x_ref[...], ws_ref[...], precision=_HI)
        + jnp.dot(hn, wn_ref[...], precision=_HI)
        + bs_ref[...]
    )
    h = jnp.where(h > 0, h, jnp.exp(h) - 1.0)
    feat = jnp.dot(h, wg_ref[...], precision=_HI)
    el_ref[...] = jnp.sum(feat * al_ref[...], axis=1, keepdims=True)
    er_ref[...] = jnp.sum(feat * ar_ref[...], axis=1, keepdims=True)
    rest_ref[...] = jnp.dot(h, wr_ref[...], precision=_HI) + bg_ref[...]
    fa_ref[...] = jnp.concatenate(
        [
            feat,
            jnp.ones((_BLK, 1), jnp.float32),
            jnp.zeros((_BLK, DF - 65), jnp.float32),
        ],
        axis=1,
    )


def _dense(x_pad, aggp, ws, wn, bs, wg, al, ar, wr, bg):
    g = NP // _BLK
    full = lambda shp: pl.BlockSpec(shp, lambda i: (0, 0))
    blk = lambda w: pl.BlockSpec((_BLK, w), lambda i: (i, 0))
    return pl.pallas_call(
        _dense_body,
        grid=(g,),
        in_specs=[
            blk(128),
            pl.BlockSpec((NC, _BLK, DA), lambda i: (0, i, 0)),
            full((128, 128)), full((128, 128)), full((1, 128)),
            full((128, 64)), full((1, 64)), full((1, 64)),
            full((128, 64)), full((1, 64)),
        ],
        out_specs=[blk(DF), blk(1), blk(1), blk(64)],
        out_shape=[
            jax.ShapeDtypeStruct((NP, DF), jnp.float32),
            jax.ShapeDtypeStruct((NP, 1), jnp.float32),
            jax.ShapeDtypeStruct((NP, 1), jnp.float32),
            jax.ShapeDtypeStruct((NP, 64), jnp.float32),
        ],
    )(x_pad, aggp, ws, wn, bs, wg, al, ar, wr, bg)


def _final_body(np_ref, rest_ref, o_ref):
    nm = np_ref[0] + np_ref[1]
    o_ref[...] = nm[:, :64] / (nm[:, 64:65] + 1e-9) + rest_ref[...]


def _final(nume, rest):
    g = NN // _BLKF
    blk = lambda w: pl.BlockSpec((_BLKF, w), lambda i: (i, 0))
    return pl.pallas_call(
        _final_body,
        grid=(g,),
        in_specs=[pl.BlockSpec((NC, _BLKF, DF), lambda i: (0, i, 0)), blk(64)],
        out_specs=blk(64),
        out_shape=jax.ShapeDtypeStruct((NN, 64), jnp.float32),
    )(nume, rest)


def kernel(x, edge_index, W_self, W_neigh, b_sage, W_gat, attn_l, attn_r, W_res, b_gat):
    src = edge_index[0]
    dst = edge_index[1]
    pad = jnp.full((EP - EE,), NP - 1, jnp.int32)
    srcf = jnp.concatenate([src, pad])
    dstf = jnp.concatenate([dst, pad])
    srcp1 = srcf[: R1ROWS * KB1].reshape(R1ROWS, KB1)
    dstp1 = dstf[: R1ROWS * KB1].reshape(R1ROWS, KB1)
    srcp2 = srcf.reshape(R2ROWS, KB2)
    dstp2 = dstf.reshape(R2ROWS, KB2)

    x_pad = jnp.pad(x, ((0, NP - NN), (0, 0)))
    x_aug = jnp.concatenate(
        [
            x_pad,
            jnp.ones((NP, 1), jnp.float32),
            jnp.zeros((NP, DA - 129), jnp.float32),
        ],
        axis=1,
    )

    aggp = _pass1(x_aug, srcp1, dstp1)

    feat_aug, el2, er2, rest = _dense(
        x_pad,
        aggp,
        W_self,
        W_neigh,
        b_sage.reshape(1, 128),
        W_gat,
        attn_l.reshape(1, 64),
        attn_r.reshape(1, 64),
        W_res,
        b_gat.reshape(1, 64),
    )

    nume = _pass2(feat_aug, el2.reshape(NP), er2.reshape(NP), srcp2, dstp2)

    return _final(nume, rest)


# speedup vs baseline: 24.3125x; 1.0168x over previous
import functools

import jax
import jax.numpy as jnp
from jax import lax
from jax.experimental import pallas as pl
from jax.experimental.pallas import tpu as pltpu
from jax.experimental.pallas import tpu_sc as plsc

NN = 10000
NP = 10240
EE = 320000
NC = 2
NS = 16
NW = NC * NS
KB1 = 64
NBA1 = 191
NBB1 = 122
R1ROWS = 16 * NBA1 + 16 * NBB1 + (NBA1 - NBB1)
KB2 = 128
NBA2 = 101
NBB2 = 56
R2ROWS = 16 * NBA2 + 16 * NBB2 + (NBA2 - NBB2)
EP = R2ROWS * KB2
DA = 136
DF = 80
RPT = NP // NS

_mesh = plsc.VectorSubcoreMesh(
    core_axis_name="c", subcore_axis_name="s", num_cores=NC, num_subcores=NS
)
_sc_params = pltpu.CompilerParams(
    use_tc_tiling_on_sc=False, needs_layout_passes=False
)


def _zero_rows(rows_v, nrows, width):
    zero = jnp.zeros((16,), jnp.float32)

    def zr(r, carry):
        for c in range(width // 16):
            rows_v[r, pl.ds(c * 16, 16)] = zero
        return carry

    lax.fori_loop(0, nrows, zr, 0)


def _pass1_body(
    x_hbm, src_hbm, dst_hbm, out_hbm, src_v, dst_v, rows_v, acc_sh, isem, gsem
):
    cid = lax.axis_index("c")
    sid = lax.axis_index("s")
    nb = jnp.where(cid == 0, NBA1, NBB1)
    base = jnp.where(cid == 0, sid * NBA1, 16 * NBA1 + sid * NBB1)
    pltpu.async_copy(src_hbm.at[pl.ds(base, NBA1)], src_v, isem)
    pltpu.async_copy(dst_hbm.at[pl.ds(base, NBA1)], dst_v, isem)
    _zero_rows(rows_v.at[0], KB1, DA)
    for b in range(RPT // KB1):
        pltpu.sync_copy(rows_v.at[0], acc_sh.at[pl.ds(sid * RPT + b * KB1, KB1)])
    pltpu.make_async_copy(src_hbm.at[pl.ds(base, NBA1)], src_v, isem).wait()
    pltpu.make_async_copy(dst_hbm.at[pl.ds(base, NBA1)], dst_v, isem).wait()
    plsc.subcore_barrier()
    pltpu.async_copy(x_hbm.at[src_v.at[0]], rows_v.at[0], gsem)

    def eb(j, carry):
        b = lax.rem(j, 2)
        pltpu.make_async_copy(x_hbm.at[src_v.at[j]], rows_v.at[b], gsem).wait()

        @pl.when(j < nb - 1)
        def _():
            pltpu.async_copy(x_hbm.at[src_v.at[j + 1]], rows_v.at[1 - b], gsem)

        pltpu.sync_copy(rows_v.at[b], acc_sh.at[dst_v.at[j]], add=True)
        return carry

    lax.fori_loop(0, nb, eb, 0)
    plsc.subcore_barrier()
    pltpu.sync_copy(
        acc_sh.at[pl.ds(sid * RPT, RPT)], out_hbm.at[cid].at[pl.ds(sid * RPT, RPT)]
    )


_pass1 = pl.kernel(
    _pass1_body,
    out_type=jax.ShapeDtypeStruct((NC, NP, DA), jnp.float32),
    mesh=_mesh,
    scratch_types=[
        pltpu.VMEM((NBA1, KB1), jnp.int32),
        pltpu.VMEM((NBA1, KB1), jnp.int32),
        pltpu.VMEM((2, KB1, DA), jnp.float32),
        pltpu.VMEM_SHARED((NP, DA), jnp.float32),
        pltpu.SemaphoreType.DMA,
        pltpu.SemaphoreType.DMA,
    ],
    compiler_params=_sc_params,
)


def _pass2_body(
    feat_hbm, el_hbm, er_hbm, src_hbm, dst_hbm, out_hbm,
    src_v, dst_v, el_tab, er_tab, rows_v, acc_sh, isem, gsem, ssem,
):
    cid = lax.axis_index("c")
    sid = lax.axis_index("s")
    nb = jnp.where(cid == 0, NBA2, NBB2)
    base = jnp.where(cid == 0, sid * NBA2, 16 * NBA2 + sid * NBB2)
    pltpu.async_copy(src_hbm.at[pl.ds(base, NBA2)], src_v, isem)
    pltpu.async_copy(dst_hbm.at[pl.ds(base, NBA2)], dst_v, isem)
    pltpu.async_copy(el_hbm, el_tab, isem)
    pltpu.async_copy(er_hbm, er_tab, isem)
    _zero_rows(rows_v.at[0], KB2, DF)
    for b in range(RPT // KB2):
        pltpu.sync_copy(rows_v.at[0], acc_sh.at[pl.ds(sid * RPT + b * KB2, KB2)])
    pltpu.make_async_copy(src_hbm.at[pl.ds(base, NBA2)], src_v, isem).wait()
    pltpu.make_async_copy(dst_hbm.at[pl.ds(base, NBA2)], dst_v, isem).wait()
    pltpu.make_async_copy(el_hbm, el_tab, isem).wait()
    pltpu.make_async_copy(er_hbm, er_tab, isem).wait()
    plsc.subcore_barrier()
    pltpu.async_copy(feat_hbm.at[src_v.at[0]], rows_v.at[0], gsem)
    pltpu.async_copy(feat_hbm.at[src_v.at[1]], rows_v.at[1], gsem)

    def eb(j, carry):
        b = lax.rem(j, 3)
        pltpu.make_async_copy(feat_hbm.at[src_v.at[j]], rows_v.at[b], gsem).wait()

        def ce(g, c2):
            s16 = src_v[j, pl.ds(g * 16, 16)]
            d16 = dst_v[j, pl.ds(g * 16, 16)]
            s = plsc.load_gather(el_tab, [s16]) + plsc.load_gather(er_tab, [d16])
            ee16 = jnp.exp(jnp.maximum(s, 0.2 * s))
            for l in range(16):
                sc = ee16[l]
                r = g * 16 + l
                for c in range(DF // 16):
                    rows_v[b, r, pl.ds(c * 16, 16)] = (
                        rows_v[b, r, pl.ds(c * 16, 16)] * sc
                    )
            return c2

        lax.fori_loop(0, KB2 // 16, ce, 0)

        @pl.when(j >= 1)
        def _():
            pltpu.make_async_copy(rows_v.at[b], acc_sh.at[dst_v.at[j]], ssem).wait()

        @pl.when(j < nb - 2)
        def _():
            pltpu.async_copy(feat_hbm.at[src_v.at[j + 2]], rows_v.at[lax.rem(j + 2, 3)], gsem)

        pltpu.async_copy(rows_v.at[b], acc_sh.at[dst_v.at[j]], ssem, add=True)
        return carry

    lax.fori_loop(0, nb, eb, 0)
    pltpu.make_async_copy(rows_v.at[0], acc_sh.at[dst_v.at[0]], ssem).wait()
    plsc.subcore_barrier()
    pltpu.sync_copy(
        acc_sh.at[pl.ds(sid * RPT, RPT)], out_hbm.at[cid].at[pl.ds(sid * RPT, RPT)]
    )


_pass2 = pl.kernel(
    _pass2_body,
    out_type=jax.ShapeDtypeStruct((NC, NP, DF), jnp.float32),
    mesh=_mesh,
    scratch_types=[
        pltpu.VMEM((NBA2, KB2), jnp.int32),
        pltpu.VMEM((NBA2, KB2), jnp.int32),
        pltpu.VMEM((NP,), jnp.float32),
        pltpu.VMEM((NP,), jnp.float32),
        pltpu.VMEM((3, KB2, DF), jnp.float32),
        pltpu.VMEM_SHARED((NP, DF), jnp.float32),
        pltpu.SemaphoreType.DMA,
        pltpu.SemaphoreType.DMA,
        pltpu.SemaphoreType.DMA,
    ],
    compiler_params=_sc_params,
)

_BLK = 512
_BLKF = 400
_HI = jax.lax.Precision.HIGHEST


def _dense_body(
    x_ref, ap_ref, ws_ref, wn_ref, bs_ref, wg_ref, al_ref, ar_ref,
    wr_ref, bg_ref, fa_ref, el_ref, er_ref, rest_ref,
):
    agg = ap_ref[0] + ap_ref[1]
    deg = jnp.maximum(agg[:, 128:129], 1.0)
    hn = agg[:, :128] / deg
    h = (
        jnp.dot(x_ref[...], ws_ref[...], precision=_HI)
        + jnp.dot(hn, wn_ref[...], precision=_HI)
        + bs_ref[...]
    )
    h = jnp.where(h > 0, h, jnp.exp(h) - 1.0)
    feat = jnp.dot(h, wg_ref[...], precision=_HI)
    el_ref[...] = jnp.sum(feat * al_ref[...], axis=1, keepdims=True)
    er_ref[...] = jnp.sum(feat * ar_ref[...], axis=1, keepdims=True)
    rest_ref[...] = jnp.dot(h, wr_ref[...], precision=_HI) + bg_ref[...]
    fa_ref[...] = jnp.concatenate(
        [
            feat,
            jnp.ones((_BLK, 1), jnp.float32),
            jnp.zeros((_BLK, DF - 65), jnp.float32),
        ],
        axis=1,
    )


def _dense(x_pad, aggp, ws, wn, bs, wg, al, ar, wr, bg):
    g = NP // _BLK
    full = lambda shp: pl.BlockSpec(shp, lambda i: (0, 0))
    blk = lambda w: pl.BlockSpec((_BLK, w), lambda i: (i, 0))
    return pl.pallas_call(
        _dense_body,
        grid=(g,),
        in_specs=[
            blk(128),
            pl.BlockSpec((NC, _BLK, DA), lambda i: (0, i, 0)),
            full((128, 128)), full((128, 128)), full((1, 128)),
            full((128, 64)), full((1, 64)), full((1, 64)),
            full((128, 64)), full((1, 64)),
        ],
        out_specs=[blk(DF), blk(1), blk(1), blk(64)],
        out_shape=[
            jax.ShapeDtypeStruct((NP, DF), jnp.float32),
            jax.ShapeDtypeStruct((NP, 1), jnp.float32),
            jax.ShapeDtypeStruct((NP, 1), jnp.float32),
            jax.ShapeDtypeStruct((NP, 64), jnp.float32),
        ],
    )(x_pad, aggp, ws, wn, bs, wg, al, ar, wr, bg)


def _final_body(np_ref, rest_ref, o_ref):
    nm = np_ref[0] + np_ref[1]
    o_ref[...] = nm[:, :64] / (nm[:, 64:65] + 1e-9) + rest_ref[...]


def _final(nume, rest):
    g = NN // _BLKF
    blk = lambda w: pl.BlockSpec((_BLKF, w), lambda i: (i, 0))
    return pl.pallas_call(
        _final_body,
        grid=(g,),
        in_specs=[pl.BlockSpec((NC, _BLKF, DF), lambda i: (0, i, 0)), blk(64)],
        out_specs=blk(64),
        out_shape=jax.ShapeDtypeStruct((NN, 64), jnp.float32),
    )(nume, rest)


def kernel(x, edge_index, W_self, W_neigh, b_sage, W_gat, attn_l, attn_r, W_res, b_gat):
    src = edge_index[0]
    dst = edge_index[1]
    pad = jnp.full((EP - EE,), NP - 1, jnp.int32)
    srcf = jnp.concatenate([src, pad])
    dstf = jnp.concatenate([dst, pad])
    srcp1 = srcf[: R1ROWS * KB1].reshape(R1ROWS, KB1)
    dstp1 = dstf[: R1ROWS * KB1].reshape(R1ROWS, KB1)
    srcp2 = srcf.reshape(R2ROWS, KB2)
    dstp2 = dstf.reshape(R2ROWS, KB2)

    x_pad = jnp.pad(x, ((0, NP - NN), (0, 0)))
    x_aug = jnp.concatenate(
        [
            x_pad,
            jnp.ones((NP, 1), jnp.float32),
            jnp.zeros((NP, DA - 129), jnp.float32),
        ],
        axis=1,
    )

    aggp = _pass1(x_aug, srcp1, dstp1)

    feat_aug, el2, er2, rest = _dense(
        x_pad,
        aggp,
        W_self,
        W_neigh,
        b_sage.reshape(1, 128),
        W_gat,
        attn_l.reshape(1, 64),
        attn_r.reshape(1, 64),
        W_res,
        b_gat.reshape(1, 64),
    )

    nume = _pass2(feat_aug, el2.reshape(NP), er2.reshape(NP), srcp2, dstp2)

    return _final(nume, rest)
